# baseline (speedup 1.0000x reference)
import jax
import jax.numpy as jnp
from jax.experimental import pallas as pl
from jax.experimental.pallas import tpu as pltpu

IN_F = 1024
BN = 3840


def _linear_block(x_ref, w_ref, b_ref, o_ref):
    j = pl.program_id(0)
    acc = jax.lax.dot_general(
        x_ref[...], w_ref[...],
        dimension_numbers=(((1,), (1,)), ((), ())),
        preferred_element_type=jnp.float32,
        precision=jax.lax.Precision.DEFAULT,
    )
    o_ref[...] = acc + b_ref[:, pl.ds(j * BN, BN)]


def kernel(x, W, b):
    batch, in_f = x.shape
    out_f = W.shape[0]
    grid = (out_f + BN - 1) // BN
    padded = grid * BN
    b2 = jnp.pad(b, (0, padded - out_f)).reshape(1, padded)
    return pl.pallas_call(
        _linear_block,
        grid=(grid,),
        in_specs=[
            pl.BlockSpec((batch, in_f), lambda j: (0, 0)),
            pl.BlockSpec((BN, in_f), lambda j: (j, 0)),
            pl.BlockSpec((1, padded), lambda j: (0, 0)),
        ],
        out_specs=pl.BlockSpec((batch, BN), lambda j: (0, j)),
        out_shape=jax.ShapeDtypeStruct((batch, out_f), jnp.float32),
        compiler_params=pltpu.CompilerParams(
            dimension_semantics=("parallel",),
        ),
    )(x, W, b2)


# speedup vs baseline: 1.0023x; 1.0023x over previous
import jax
import jax.numpy as jnp
from jax.experimental import pallas as pl
from jax.experimental.pallas import tpu as pltpu

IN_F = 1024
BN = 3328


def _linear_block(x_ref, w_ref, b_ref, o_ref):
    j = pl.program_id(0)
    acc = jax.lax.dot_general(
        x_ref[...], w_ref[...],
        dimension_numbers=(((1,), (1,)), ((), ())),
        preferred_element_type=jnp.float32,
        precision=jax.lax.Precision.DEFAULT,
    )
    o_ref[...] = acc + b_ref[:, pl.ds(j * BN, BN)]


def kernel(x, W, b):
    batch, in_f = x.shape
    out_f = W.shape[0]
    grid = (out_f + BN - 1) // BN
    padded = grid * BN
    b2 = jnp.pad(b, (0, padded - out_f)).reshape(1, padded)
    return pl.pallas_call(
        _linear_block,
        grid=(grid,),
        in_specs=[
            pl.BlockSpec((batch, in_f), lambda j: (0, 0)),
            pl.BlockSpec((BN, in_f), lambda j: (j, 0)),
            pl.BlockSpec((1, padded), lambda j: (0, 0)),
        ],
        out_specs=pl.BlockSpec((batch, BN), lambda j: (0, j)),
        out_shape=jax.ShapeDtypeStruct((batch, out_f), jnp.float32),
        compiler_params=pltpu.CompilerParams(
            dimension_semantics=("parallel",),
        ),
    )(x, W, b2)


# speedup vs baseline: 1.0192x; 1.0169x over previous
import jax
import jax.numpy as jnp
from jax.experimental import pallas as pl
from jax.experimental.pallas import tpu as pltpu

IN_F = 1024
BN = 3072


def _linear_block(x_ref, w_ref, b_ref, o_ref):
    j = pl.program_id(0)
    acc = jax.lax.dot_general(
        x_ref[...], w_ref[...],
        dimension_numbers=(((1,), (1,)), ((), ())),
        preferred_element_type=jnp.float32,
        precision=jax.lax.Precision.DEFAULT,
    )
    o_ref[...] = acc + b_ref[:, pl.ds(j * BN, BN)]


def kernel(x, W, b):
    batch, in_f = x.shape
    out_f = W.shape[0]
    grid = (out_f + BN - 1) // BN
    padded = grid * BN
    b2 = jnp.pad(b, (0, padded - out_f)).reshape(1, padded)
    return pl.pallas_call(
        _linear_block,
        grid=(grid,),
        in_specs=[
            pl.BlockSpec((batch, in_f), lambda j: (0, 0)),
            pl.BlockSpec((BN, in_f), lambda j: (j, 0)),
            pl.BlockSpec((1, padded), lambda j: (0, 0)),
        ],
        out_specs=pl.BlockSpec((batch, BN), lambda j: (0, j)),
        out_shape=jax.ShapeDtypeStruct((batch, out_f), jnp.float32),
        compiler_params=pltpu.CompilerParams(
            dimension_semantics=("parallel",),
        ),
    )(x, W, b2)


# speedup vs baseline: 1.0225x; 1.0032x over previous
import jax
import jax.numpy as jnp
from jax.experimental import pallas as pl
from jax.experimental.pallas import tpu as pltpu

IN_F = 1024
BN = 2816


def _linear_block(x_ref, w_ref, b_ref, o_ref):
    j = pl.program_id(0)
    acc = jax.lax.dot_general(
        x_ref[...], w_ref[...],
        dimension_numbers=(((1,), (1,)), ((), ())),
        preferred_element_type=jnp.float32,
        precision=jax.lax.Precision.DEFAULT,
    )
    o_ref[...] = acc + b_ref[:, pl.ds(j * BN, BN)]


def kernel(x, W, b):
    batch, in_f = x.shape
    out_f = W.shape[0]
    grid = (out_f + BN - 1) // BN
    padded = grid * BN
    b2 = jnp.pad(b, (0, padded - out_f)).reshape(1, padded)
    return pl.pallas_call(
        _linear_block,
        grid=(grid,),
        in_specs=[
            pl.BlockSpec((batch, in_f), lambda j: (0, 0)),
            pl.BlockSpec((BN, in_f), lambda j: (j, 0)),
            pl.BlockSpec((1, padded), lambda j: (0, 0)),
        ],
        out_specs=pl.BlockSpec((batch, BN), lambda j: (0, j)),
        out_shape=jax.ShapeDtypeStruct((batch, out_f), jnp.float32),
        compiler_params=pltpu.CompilerParams(
            dimension_semantics=("parallel",),
        ),
    )(x, W, b2)
